# baseline (speedup 1.0000x reference)
import jax
import jax.numpy as jnp
from jax import lax
from jax.experimental import pallas as pl
from jax.experimental.pallas import tpu as pltpu

_CLAMP = 1.5


def _spline_kernel(x_ref, t0_ref, t1_ref, o_ref):
    T0 = t0_ref[...]
    T1 = t1_ref[...]
    a15 = T1[:, 36:37]
    b15 = T1[:, 37:38]
    g = T1[:, 38:39]
    bias = T1[:, 39:40]

    x = x_ref[...]
    kk = 15.5
    lim = _CLAMP * kk
    y = lax.clamp(-lim, x * a15 + b15, lim)
    u = y + kk
    fi = jnp.floor(u)
    t = u - fi
    m = (lax.clamp(-2.0, fi, 33.0) + 2.0).astype(jnp.int32)
    q0 = jnp.take_along_axis(T0, m, axis=1)
    q1 = jnp.take_along_axis(T0, m + 36, axis=1)
    q2 = jnp.take_along_axis(T0, m + 72, axis=1)
    q3 = jnp.take_along_axis(T1, m, axis=1)
    s = ((q3 * t + q2) * t + q1) * t + q0
    o_ref[...] = g * x + s + bias


def kernel(x, a, b, alpha, id_gain, bias):
    B, C, H, W = x.shape
    K = alpha.shape[-1]
    R = C * H
    x2 = x.reshape(B * R, W)

    pad_idx = jnp.clip(jnp.arange(40) - 3, 0, K - 1)
    ap = alpha[:, pad_idx]
    A0 = ap[:, 0:36]
    A1 = ap[:, 1:37]
    A2 = ap[:, 2:38]
    A3 = ap[:, 3:39]
    p0 = (A0 + 4.0 * A1 + A2) * (1.0 / 6.0)
    p1 = (A2 - A0) * 0.5
    p2 = (A0 + A2) * 0.5 - A1
    p3 = (A3 - A0 + 3.0 * (A1 - A2)) * (1.0 / 6.0)
    kk = 0.5 * (K - 1)
    T0 = jnp.concatenate([p0, p1, p2], axis=1)
    T1 = jnp.concatenate(
        [p3, (a * kk)[:, None], (b * kk)[:, None], id_gain[:, None],
         bias[:, None]],
        axis=1,
    )
    T0r = jnp.repeat(T0, H, axis=0)
    T1r = jnp.repeat(T1, H, axis=0)

    S = 8
    RB = R // S
    out = pl.pallas_call(
        _spline_kernel,
        grid=(S, B),
        in_specs=[
            pl.BlockSpec((RB, W), lambda i, j: (j * S + i, 0)),
            pl.BlockSpec((RB, 108), lambda i, j: (i, 0)),
            pl.BlockSpec((RB, 40), lambda i, j: (i, 0)),
        ],
        out_specs=pl.BlockSpec((RB, W), lambda i, j: (j * S + i, 0)),
        out_shape=jax.ShapeDtypeStruct((B * R, W), jnp.float32),
        compiler_params=pltpu.CompilerParams(
            dimension_semantics=("parallel", "arbitrary"),
        ),
    )(x2, T0r, T1r)
    return out.reshape(B, C, H, W)


# speedup vs baseline: 1.1677x; 1.1677x over previous
import jax
import jax.numpy as jnp
from jax import lax
from jax.experimental import pallas as pl
from jax.experimental.pallas import tpu as pltpu


def _spline_kernel(x_ref, w_ref, o_ref):
    w = w_ref[...]
    A0 = w[:, 0:36]
    A1 = w[:, 1:37]
    A2 = w[:, 2:38]
    A3 = w[:, 3:39]
    bias = w[:, 43:44]
    p0 = (A0 + 4.0 * A1 + A2) * (1.0 / 6.0) + bias
    p1 = (A2 - A0) * 0.5
    p2 = (A0 + A2) * 0.5 - A1
    p3 = (A3 - A0 + 3.0 * (A1 - A2)) * (1.0 / 6.0)

    a15 = w[:, 40:41] * 15.5
    b2 = w[:, 41:42] * 15.5 + 17.5
    g = w[:, 42:43]

    x = x_ref[...]
    u2 = x * a15 + b2
    uc = jnp.minimum(jnp.maximum(u2, 0.0), 35.5)
    fi = jnp.floor(uc)
    t = uc - fi
    m = jnp.round(fi).astype(jnp.int32)
    q0 = jnp.take_along_axis(p0, m, axis=1)
    q1 = jnp.take_along_axis(p1, m, axis=1)
    q2 = jnp.take_along_axis(p2, m, axis=1)
    q3 = jnp.take_along_axis(p3, m, axis=1)
    s = ((q3 * t + q2) * t + q1) * t + q0
    o_ref[...] = g * x + s


def kernel(x, a, b, alpha, id_gain, bias):
    B, C, H, W = x.shape
    K = alpha.shape[-1]
    HW = H * W
    x2 = x.reshape(B * C, HW)

    pad_idx = jnp.clip(jnp.arange(40) - 3, 0, K - 1)
    alpha_pad = alpha[:, pad_idx]
    w = jnp.concatenate(
        [alpha_pad, a[:, None], b[:, None], id_gain[:, None], bias[:, None]],
        axis=1,
    )

    LB = HW // 2
    grid = (B, HW // LB)
    out = pl.pallas_call(
        _spline_kernel,
        grid=grid,
        in_specs=[
            pl.BlockSpec((C, LB), lambda i, j: (i, j)),
            pl.BlockSpec((C, 44), lambda i, j: (0, 0)),
        ],
        out_specs=pl.BlockSpec((C, LB), lambda i, j: (i, j)),
        out_shape=jax.ShapeDtypeStruct((B * C, HW), jnp.float32),
        compiler_params=pltpu.CompilerParams(
            dimension_semantics=("parallel", "arbitrary"),
        ),
    )(x2, w)
    return out.reshape(B, C, H, W)


# speedup vs baseline: 1.2046x; 1.0315x over previous
import jax
import jax.numpy as jnp
from jax import lax
from jax.experimental import pallas as pl
from jax.experimental.pallas import tpu as pltpu


def _spline_kernel(x_ref, w_ref, o_ref):
    w = w_ref[...]
    A0 = w[:, 0:36]
    A1 = w[:, 1:37]
    A2 = w[:, 2:38]
    A3 = w[:, 3:39]
    bias = w[:, 43:44]
    p0 = (A0 + 4.0 * A1 + A2) * (1.0 / 6.0) + bias
    p1 = (A2 - A0) * 0.5
    p2 = (A0 + A2) * 0.5 - A1
    p3 = (A3 - A0 + 3.0 * (A1 - A2)) * (1.0 / 6.0)

    a15 = w[:, 40:41] * 15.5
    b2 = w[:, 41:42] * 15.5 + 17.5
    g = w[:, 42:43]

    L = x_ref.shape[1]
    CK = 128
    for k in range(L // CK):
        x = x_ref[:, k * CK:(k + 1) * CK]
        u2 = x * a15 + b2
        uc = jnp.minimum(jnp.maximum(u2, 0.0), 35.5)
        fi = jnp.floor(uc)
        t = uc - fi
        m = jnp.round(fi).astype(jnp.int32)
        q0 = jnp.take_along_axis(p0, m, axis=1)
        q1 = jnp.take_along_axis(p1, m, axis=1)
        q2 = jnp.take_along_axis(p2, m, axis=1)
        q3 = jnp.take_along_axis(p3, m, axis=1)
        s = ((q3 * t + q2) * t + q1) * t + q0
        o_ref[:, k * CK:(k + 1) * CK] = g * x + s


def kernel(x, a, b, alpha, id_gain, bias):
    B, C, H, W = x.shape
    K = alpha.shape[-1]
    HW = H * W
    x2 = x.reshape(B * C, HW)

    pad_idx = jnp.clip(jnp.arange(40) - 3, 0, K - 1)
    alpha_pad = alpha[:, pad_idx]
    w = jnp.concatenate(
        [alpha_pad, a[:, None], b[:, None], id_gain[:, None], bias[:, None]],
        axis=1,
    )

    LB = HW // 2
    grid = (B, HW // LB)
    out = pl.pallas_call(
        _spline_kernel,
        grid=grid,
        in_specs=[
            pl.BlockSpec((C, LB), lambda i, j: (i, j)),
            pl.BlockSpec((C, 44), lambda i, j: (0, 0)),
        ],
        out_specs=pl.BlockSpec((C, LB), lambda i, j: (i, j)),
        out_shape=jax.ShapeDtypeStruct((B * C, HW), jnp.float32),
        compiler_params=pltpu.CompilerParams(
            dimension_semantics=("parallel", "arbitrary"),
        ),
    )(x2, w)
    return out.reshape(B, C, H, W)
